# baseline (speedup 1.0000x reference)
import functools

import jax
import jax.numpy as jnp
from jax.experimental import pallas as pl
from jax.experimental.pallas import tpu as pltpu

_BLOCK_N = 2048
_K = 8
_NBUF = 3


def _compute(x, w1, b1, w2, b2):
    ht = jax.lax.dot_general(
        w1, x, (((0,), (1,)), ((), ())),
        preferred_element_type=jnp.float32,
    ) + b1
    ht = 0.5 * ht * (1.0 + jax.lax.erf(ht * 0.7071067811865476))
    lt = jax.lax.dot_general(
        w2, ht, (((0,), (0,)), ((), ())),
        preferred_element_type=jnp.float32,
    ) + b2
    m = jnp.max(lt, axis=0, keepdims=True)
    e = jnp.exp(lt - m)
    return e / jnp.sum(e, axis=0, keepdims=True)


def _body(x_hbm, w1_hbm, b1_hbm, w2_hbm, b2_hbm, o_hbm, *, n, d, h_dim, q):
    spc = n // _BLOCK_N // _K

    def run(w1_v, b1_v, w2_v, b2_v, x_bufs, o_bufs, w_sem, in_sems, out_sems):
        for ref_h, ref_v in ((w1_hbm, w1_v), (b1_hbm, b1_v),
                             (w2_hbm, w2_v), (b2_hbm, b2_v)):
            cp = pltpu.make_async_copy(ref_h, ref_v, w_sem)
            cp.start()
            cp.wait()
        w1 = w1_v[...]
        b1 = b1_v[...]
        w2 = w2_v[...]
        b2 = b2_v[...]

        def in_copy(k, j, slot):
            row0 = (k * spc + j) * _BLOCK_N
            return pltpu.make_async_copy(
                x_hbm.at[pl.ds(row0, _BLOCK_N), :],
                x_bufs.at[k, slot],
                in_sems.at[k, slot],
            )

        def out_copy(k, j, slot):
            col0 = (k * spc + j) * _BLOCK_N
            return pltpu.make_async_copy(
                o_bufs.at[k, slot],
                o_hbm.at[:, pl.ds(col0, _BLOCK_N)],
                out_sems.at[k, slot],
            )

        for j in range(_NBUF - 1):
            for k in range(_K):
                in_copy(k, j, j).start()

        def step(j, carry):
            del carry
            slot = jax.lax.rem(j, _NBUF)
            oslot = jax.lax.rem(j, 2)
            for k in range(_K):
                in_copy(k, j, slot).wait()

                @pl.when(j >= 2)
                def _():
                    out_copy(k, j - 2, oslot).wait()

                p = _compute(x_bufs[k, slot], w1, b1, w2, b2)
                o_bufs[k, oslot] = p
                out_copy(k, j, oslot).start()

                @pl.when(j + _NBUF - 1 < spc)
                def _():
                    nslot = jax.lax.rem(j + _NBUF - 1, _NBUF)
                    in_copy(k, j + _NBUF - 1, nslot).start()
            return 0

        jax.lax.fori_loop(0, spc, step, 0)

        for k in range(_K):
            for j in (spc - 2, spc - 1):
                out_copy(k, j, j % 2).wait()

    pl.run_scoped(
        run,
        pltpu.VMEM((d, h_dim), jnp.float32),
        pltpu.VMEM((h_dim, 1), jnp.float32),
        pltpu.VMEM((h_dim, q), jnp.float32),
        pltpu.VMEM((q, 1), jnp.float32),
        pltpu.VMEM((_K, _NBUF, _BLOCK_N, d), jnp.float32),
        pltpu.VMEM((_K, 2, q, _BLOCK_N), jnp.float32),
        pltpu.SemaphoreType.DMA,
        pltpu.SemaphoreType.DMA((_K, _NBUF)),
        pltpu.SemaphoreType.DMA((_K, 2)),
    )


def kernel(batch_x, W1, b1, W2, b2):
    n, d = batch_x.shape
    h_dim = W1.shape[1]
    q = W2.shape[1]
    mesh = pltpu.create_tensorcore_mesh("core")
    body = functools.partial(_body, n=n, d=d, h_dim=h_dim, q=q)
    out_t = pl.kernel(
        body,
        out_type=jax.ShapeDtypeStruct((q, n), jnp.float32),
        mesh=mesh,
    )(batch_x, W1, b1.reshape(h_dim, 1), W2, b2.reshape(q, 1))
    return out_t.T


# speedup vs baseline: 1.0946x; 1.0946x over previous
import functools

import jax
import jax.numpy as jnp
from jax.experimental import pallas as pl
from jax.experimental.pallas import tpu as pltpu

_BLOCK_N = 4096
_K = 4
_NBUF = 4


def _compute(x, w1, b1, w2, b2):
    ht = jax.lax.dot_general(
        w1, x, (((0,), (1,)), ((), ())),
        preferred_element_type=jnp.float32,
    ) + b1
    ht = 0.5 * ht * (1.0 + jax.lax.erf(ht * 0.7071067811865476))
    lt = jax.lax.dot_general(
        w2, ht, (((0,), (0,)), ((), ())),
        preferred_element_type=jnp.float32,
    ) + b2
    m = jnp.max(lt, axis=0, keepdims=True)
    e = jnp.exp(lt - m)
    return e / jnp.sum(e, axis=0, keepdims=True)


def _body(x_hbm, w1_hbm, b1_hbm, w2_hbm, b2_hbm, o_hbm, *, n, d, h_dim, q):
    spc = n // _BLOCK_N // _K

    def run(w1_v, b1_v, w2_v, b2_v, x_bufs, o_bufs, w_sem, in_sems, out_sems):
        for ref_h, ref_v in ((w1_hbm, w1_v), (b1_hbm, b1_v),
                             (w2_hbm, w2_v), (b2_hbm, b2_v)):
            cp = pltpu.make_async_copy(ref_h, ref_v, w_sem)
            cp.start()
            cp.wait()
        w1 = w1_v[...]
        b1 = b1_v[...]
        w2 = w2_v[...]
        b2 = b2_v[...]

        def in_copy(k, j, slot):
            row0 = (k * spc + j) * _BLOCK_N
            return pltpu.make_async_copy(
                x_hbm.at[pl.ds(row0, _BLOCK_N), :],
                x_bufs.at[k, slot],
                in_sems.at[k, slot],
            )

        def out_copy(k, j, slot):
            col0 = (k * spc + j) * _BLOCK_N
            return pltpu.make_async_copy(
                o_bufs.at[k, slot],
                o_hbm.at[:, pl.ds(col0, _BLOCK_N)],
                out_sems.at[k, slot],
            )

        for j in range(_NBUF - 1):
            for k in range(_K):
                in_copy(k, j, j).start()

        def step(j, carry):
            del carry
            slot = jax.lax.rem(j, _NBUF)
            oslot = jax.lax.rem(j, 2)
            for k in range(_K):
                in_copy(k, j, slot).wait()

                @pl.when(j >= 2)
                def _():
                    out_copy(k, j - 2, oslot).wait()

                p = _compute(x_bufs[k, slot], w1, b1, w2, b2)
                o_bufs[k, oslot] = p
                out_copy(k, j, oslot).start()

                @pl.when(j + _NBUF - 1 < spc)
                def _():
                    nslot = jax.lax.rem(j + _NBUF - 1, _NBUF)
                    in_copy(k, j + _NBUF - 1, nslot).start()
            return 0

        jax.lax.fori_loop(0, spc, step, 0)

        for k in range(_K):
            for j in (spc - 2, spc - 1):
                out_copy(k, j, j % 2).wait()

    pl.run_scoped(
        run,
        pltpu.VMEM((d, h_dim), jnp.float32),
        pltpu.VMEM((h_dim, 1), jnp.float32),
        pltpu.VMEM((h_dim, q), jnp.float32),
        pltpu.VMEM((q, 1), jnp.float32),
        pltpu.VMEM((_K, _NBUF, _BLOCK_N, d), jnp.float32),
        pltpu.VMEM((_K, 2, q, _BLOCK_N), jnp.float32),
        pltpu.SemaphoreType.DMA,
        pltpu.SemaphoreType.DMA((_K, _NBUF)),
        pltpu.SemaphoreType.DMA((_K, 2)),
    )


def kernel(batch_x, W1, b1, W2, b2):
    n, d = batch_x.shape
    h_dim = W1.shape[1]
    q = W2.shape[1]
    mesh = pltpu.create_tensorcore_mesh("core")
    body = functools.partial(_body, n=n, d=d, h_dim=h_dim, q=q)
    out_t = pl.kernel(
        body,
        out_type=jax.ShapeDtypeStruct((q, n), jnp.float32),
        mesh=mesh,
    )(batch_x, W1, b1.reshape(h_dim, 1), W2, b2.reshape(q, 1))
    return out_t.T


# speedup vs baseline: 1.1007x; 1.0056x over previous
import functools

import jax
import jax.numpy as jnp
from jax.experimental import pallas as pl
from jax.experimental.pallas import tpu as pltpu

_BLOCK_N = 4096
_K = 4
_NBUF = 3


def _compute(x, w1, b1, w2, b2):
    ht = jax.lax.dot_general(
        w1, x, (((0,), (1,)), ((), ())),
        preferred_element_type=jnp.float32,
    ) + b1
    ht = 0.5 * ht * (1.0 + jax.lax.erf(ht * 0.7071067811865476))
    lt = jax.lax.dot_general(
        w2, ht, (((0,), (0,)), ((), ())),
        preferred_element_type=jnp.float32,
    ) + b2
    m = jnp.max(lt, axis=0, keepdims=True)
    e = jnp.exp(lt - m)
    return e / jnp.sum(e, axis=0, keepdims=True)


def _body(x_hbm, w1_hbm, b1_hbm, w2_hbm, b2_hbm, o_hbm, *, n, d, h_dim, q):
    spc = n // _BLOCK_N // _K

    def run(w1_v, b1_v, w2_v, b2_v, x_bufs, o_bufs, w_sem, in_sems, out_sems):
        for ref_h, ref_v in ((w1_hbm, w1_v), (b1_hbm, b1_v),
                             (w2_hbm, w2_v), (b2_hbm, b2_v)):
            cp = pltpu.make_async_copy(ref_h, ref_v, w_sem)
            cp.start()
            cp.wait()
        w1 = w1_v[...]
        b1 = b1_v[...]
        w2 = w2_v[...]
        b2 = b2_v[...]

        def in_copy(k, j, slot):
            row0 = (k * spc + j) * _BLOCK_N
            return pltpu.make_async_copy(
                x_hbm.at[pl.ds(row0, _BLOCK_N), :],
                x_bufs.at[k, slot],
                in_sems.at[k, slot],
            )

        def out_copy(k, j, slot):
            col0 = (k * spc + j) * _BLOCK_N
            return pltpu.make_async_copy(
                o_bufs.at[k, slot],
                o_hbm.at[:, pl.ds(col0, _BLOCK_N)],
                out_sems.at[k, slot],
            )

        for j in range(_NBUF - 1):
            for k in range(_K):
                in_copy(k, j, j).start()

        def step(j, carry):
            del carry
            slot = jax.lax.rem(j, _NBUF)
            oslot = jax.lax.rem(j, 2)
            for k in range(_K):
                in_copy(k, j, slot).wait()

                @pl.when(j >= 2)
                def _():
                    out_copy(k, j - 2, oslot).wait()

                p = _compute(x_bufs[k, slot], w1, b1, w2, b2)
                o_bufs[k, oslot] = p
                out_copy(k, j, oslot).start()

                @pl.when(j + _NBUF - 1 < spc)
                def _():
                    nslot = jax.lax.rem(j + _NBUF - 1, _NBUF)
                    in_copy(k, j + _NBUF - 1, nslot).start()
            return 0

        jax.lax.fori_loop(0, spc, step, 0)

        for k in range(_K):
            for j in (spc - 2, spc - 1):
                out_copy(k, j, j % 2).wait()

    pl.run_scoped(
        run,
        pltpu.VMEM((d, h_dim), jnp.float32),
        pltpu.VMEM((h_dim, 1), jnp.float32),
        pltpu.VMEM((h_dim, q), jnp.float32),
        pltpu.VMEM((q, 1), jnp.float32),
        pltpu.VMEM((_K, _NBUF, _BLOCK_N, d), jnp.float32),
        pltpu.VMEM((_K, 2, q, _BLOCK_N), jnp.float32),
        pltpu.SemaphoreType.DMA,
        pltpu.SemaphoreType.DMA((_K, _NBUF)),
        pltpu.SemaphoreType.DMA((_K, 2)),
    )


def kernel(batch_x, W1, b1, W2, b2):
    n, d = batch_x.shape
    h_dim = W1.shape[1]
    q = W2.shape[1]
    mesh = pltpu.create_tensorcore_mesh("core")
    body = functools.partial(_body, n=n, d=d, h_dim=h_dim, q=q)
    out_t = pl.kernel(
        body,
        out_type=jax.ShapeDtypeStruct((q, n), jnp.float32),
        mesh=mesh,
    )(batch_x, W1, b1.reshape(h_dim, 1), W2, b2.reshape(q, 1))
    return out_t.T
